# baseline (device time: 47001 ns/iter reference)
import jax
import jax.numpy as jnp
from jax import lax
from jax.experimental import pallas as pl
from jax.experimental.pallas import tpu as pltpu

N_DEV = 4


def kernel(t, W):
    m, k = t.shape
    kk, n = W.shape

    def body(t_ref, w_ref, out_ref, comm_ref, send_sems, recv_sems):
        my_pos = lax.axis_index("i")
        left = (my_pos - 1) % N_DEV
        right = (my_pos + 1) % N_DEV

        barrier_sem = pltpu.get_barrier_semaphore()
        for nbr in [left, right]:
            pl.semaphore_signal(
                barrier_sem, inc=1,
                device_id=(nbr,), device_id_type=pl.DeviceIdType.MESH,
            )
        pl.semaphore_wait(barrier_sem, 2)

        comm_ref[0, :, :] = t_ref[:, :].astype(jnp.bfloat16)

        for h in range(N_DEV - 1):
            rdma = pltpu.make_async_remote_copy(
                src_ref=comm_ref.at[h],
                dst_ref=comm_ref.at[h + 1],
                send_sem=send_sems.at[h],
                recv_sem=recv_sems.at[h],
                device_id=(right,),
                device_id_type=pl.DeviceIdType.MESH,
            )
            rdma.start()
            rdma.wait()

        acc = jnp.sum(comm_ref[:, :, :].astype(jnp.float32), axis=0)
        out_ref[:, :] = jnp.dot(
            acc.astype(jnp.bfloat16),
            w_ref[:, :].astype(jnp.bfloat16),
            preferred_element_type=jnp.float32,
        )

    return pl.pallas_call(
        body,
        out_shape=jax.ShapeDtypeStruct((m, n), jnp.float32),
        in_specs=[
            pl.BlockSpec(memory_space=pltpu.VMEM),
            pl.BlockSpec(memory_space=pltpu.VMEM),
        ],
        out_specs=pl.BlockSpec(memory_space=pltpu.VMEM),
        scratch_shapes=[
            pltpu.VMEM((N_DEV, m, k), jnp.bfloat16),
            pltpu.SemaphoreType.DMA((N_DEV - 1,)),
            pltpu.SemaphoreType.DMA((N_DEV - 1,)),
        ],
        compiler_params=pltpu.CompilerParams(collective_id=0),
    )(t, W)


# device time: 23002 ns/iter; 2.0433x vs baseline; 2.0433x over previous
import jax
import jax.numpy as jnp
from jax import lax
from jax.experimental import pallas as pl
from jax.experimental.pallas import tpu as pltpu

N_DEV = 4


def kernel(t, W):
    m, k = t.shape
    _, n = W.shape
    S = m // N_DEV

    def body(t_ref, w_ref, out_ref, tb_ref, rs_ref, ag_ref,
             rs_send, rs_recv, ag_send, ag_recv):
        my_pos = lax.axis_index("i")

        barrier_sem = pltpu.get_barrier_semaphore()
        for off in range(1, N_DEV):
            peer = lax.rem(my_pos + off, N_DEV)
            pl.semaphore_signal(
                barrier_sem, inc=1,
                device_id=(peer,), device_id_type=pl.DeviceIdType.MESH,
            )
        pl.semaphore_wait(barrier_sem, N_DEV - 1)

        tb_ref[...] = t_ref[...].astype(jnp.bfloat16).reshape(N_DEV, S, k)

        rs_rdmas = []
        for off in range(1, N_DEV):
            peer = lax.rem(my_pos + off, N_DEV)
            rdma = pltpu.make_async_remote_copy(
                src_ref=tb_ref.at[peer],
                dst_ref=rs_ref.at[my_pos],
                send_sem=rs_send.at[peer],
                recv_sem=rs_recv.at[my_pos],
                device_id=(peer,),
                device_id_type=pl.DeviceIdType.MESH,
            )
            rdma.start()
            rs_rdmas.append(rdma)

        rs_ref[pl.ds(my_pos, 1), :, :] = tb_ref[pl.ds(my_pos, 1), :, :]

        for off in range(1, N_DEV):
            peer = lax.rem(my_pos + off, N_DEV)
            recv = pltpu.make_async_remote_copy(
                src_ref=tb_ref.at[peer],
                dst_ref=rs_ref.at[peer],
                send_sem=rs_send.at[peer],
                recv_sem=rs_recv.at[peer],
                device_id=(peer,),
                device_id_type=pl.DeviceIdType.MESH,
            )
            recv.wait_recv()
        for rdma in rs_rdmas:
            rdma.wait_send()

        acc = jnp.sum(rs_ref[...].astype(jnp.float32), axis=0)
        y = jnp.dot(
            acc.astype(jnp.bfloat16),
            w_ref[...].astype(jnp.bfloat16),
            preferred_element_type=jnp.float32,
        )
        ag_ref[pl.ds(my_pos, 1), :, :] = y[None].astype(jnp.bfloat16)

        ag_rdmas = []
        for off in range(1, N_DEV):
            peer = lax.rem(my_pos + off, N_DEV)
            rdma = pltpu.make_async_remote_copy(
                src_ref=ag_ref.at[my_pos],
                dst_ref=ag_ref.at[my_pos],
                send_sem=ag_send.at[peer],
                recv_sem=ag_recv.at[my_pos],
                device_id=(peer,),
                device_id_type=pl.DeviceIdType.MESH,
            )
            rdma.start()
            ag_rdmas.append(rdma)

        out_ref[pl.ds(my_pos * S, S), :] = y

        for off in range(1, N_DEV):
            peer = lax.rem(my_pos + off, N_DEV)
            recv = pltpu.make_async_remote_copy(
                src_ref=ag_ref.at[peer],
                dst_ref=ag_ref.at[peer],
                send_sem=ag_send.at[peer],
                recv_sem=ag_recv.at[peer],
                device_id=(peer,),
                device_id_type=pl.DeviceIdType.MESH,
            )
            recv.wait_recv()
            out_ref[pl.ds(peer * S, S), :] = ag_ref[pl.ds(peer, 1), :, :][0].astype(
                jnp.float32
            )
        for rdma in ag_rdmas:
            rdma.wait_send()

    return pl.pallas_call(
        body,
        out_shape=jax.ShapeDtypeStruct((m, n), jnp.float32),
        in_specs=[
            pl.BlockSpec(memory_space=pltpu.VMEM),
            pl.BlockSpec(memory_space=pltpu.VMEM),
        ],
        out_specs=pl.BlockSpec(memory_space=pltpu.VMEM),
        scratch_shapes=[
            pltpu.VMEM((N_DEV, S, k), jnp.bfloat16),
            pltpu.VMEM((N_DEV, S, k), jnp.bfloat16),
            pltpu.VMEM((N_DEV, S, n), jnp.bfloat16),
            pltpu.SemaphoreType.DMA((N_DEV,)),
            pltpu.SemaphoreType.DMA((N_DEV,)),
            pltpu.SemaphoreType.DMA((N_DEV,)),
            pltpu.SemaphoreType.DMA((N_DEV,)),
        ],
        compiler_params=pltpu.CompilerParams(collective_id=0),
    )(t, W)


# device time: 20371 ns/iter; 2.3073x vs baseline; 1.1292x over previous
import jax
import jax.numpy as jnp
from jax import lax
from jax.experimental import pallas as pl
from jax.experimental.pallas import tpu as pltpu

N_DEV = 4
H = 2


def kernel(t, W):
    m, k = t.shape
    _, n = W.shape
    S = m // N_DEV
    Sh = S // H

    def body(t_ref, w_ref, out_ref, rs_ref, rs_send, rs_recv,
             ag_send, ag_recv):
        my_pos = lax.axis_index("i")

        barrier_sem = pltpu.get_barrier_semaphore()
        for off in range(1, N_DEV):
            peer = lax.rem(my_pos + off, N_DEV)
            pl.semaphore_signal(
                barrier_sem, inc=1,
                device_id=(peer,), device_id_type=pl.DeviceIdType.MESH,
            )
        pl.semaphore_wait(barrier_sem, N_DEV - 1)

        rs_rdmas = []
        for h in range(H):
            for off in range(1, N_DEV):
                peer = lax.rem(my_pos + off, N_DEV)
                rdma = pltpu.make_async_remote_copy(
                    src_ref=t_ref.at[pl.ds(peer * S + h * Sh, Sh), :],
                    dst_ref=rs_ref.at[my_pos, pl.ds(h * Sh, Sh), :],
                    send_sem=rs_send.at[peer, h],
                    recv_sem=rs_recv.at[my_pos, h],
                    device_id=(peer,),
                    device_id_type=pl.DeviceIdType.MESH,
                )
                rdma.start()
                rs_rdmas.append(rdma)

        ag_rdmas = []
        for h in range(H):
            for off in range(1, N_DEV):
                peer = lax.rem(my_pos + off, N_DEV)
                recv = pltpu.make_async_remote_copy(
                    src_ref=t_ref.at[pl.ds(peer * S + h * Sh, Sh), :],
                    dst_ref=rs_ref.at[peer, pl.ds(h * Sh, Sh), :],
                    send_sem=rs_send.at[peer, h],
                    recv_sem=rs_recv.at[peer, h],
                    device_id=(peer,),
                    device_id_type=pl.DeviceIdType.MESH,
                )
                recv.wait_recv()

            acc = t_ref[pl.ds(my_pos * S + h * Sh, Sh), :].astype(jnp.float32)
            for off in range(1, N_DEV):
                peer = lax.rem(my_pos + off, N_DEV)
                acc = acc + rs_ref[pl.ds(peer, 1), pl.ds(h * Sh, Sh), :][0].astype(
                    jnp.float32
                )
            y_h = jnp.dot(
                acc.astype(jnp.bfloat16), w_ref[...],
                preferred_element_type=jnp.float32,
            )

            row0 = my_pos * S + h * Sh
            out_ref[pl.ds(row0, Sh), :] = y_h.astype(jnp.bfloat16)

            for off in range(1, N_DEV):
                peer = lax.rem(my_pos + off, N_DEV)
                rdma = pltpu.make_async_remote_copy(
                    src_ref=out_ref.at[pl.ds(row0, Sh), :],
                    dst_ref=out_ref.at[pl.ds(row0, Sh), :],
                    send_sem=ag_send.at[peer, h],
                    recv_sem=ag_recv.at[my_pos, h],
                    device_id=(peer,),
                    device_id_type=pl.DeviceIdType.MESH,
                )
                rdma.start()
                ag_rdmas.append(rdma)

        for h in range(H):
            for off in range(1, N_DEV):
                peer = lax.rem(my_pos + off, N_DEV)
                prow0 = peer * S + h * Sh
                recv = pltpu.make_async_remote_copy(
                    src_ref=out_ref.at[pl.ds(prow0, Sh), :],
                    dst_ref=out_ref.at[pl.ds(prow0, Sh), :],
                    send_sem=ag_send.at[peer, h],
                    recv_sem=ag_recv.at[peer, h],
                    device_id=(peer,),
                    device_id_type=pl.DeviceIdType.MESH,
                )
                recv.wait_recv()
        for rdma in rs_rdmas:
            rdma.wait_send()
        for rdma in ag_rdmas:
            rdma.wait_send()

    tb = t.astype(jnp.bfloat16)
    wb = W.astype(jnp.bfloat16)
    return pl.pallas_call(
        body,
        out_shape=jax.ShapeDtypeStruct((m, n), jnp.bfloat16),
        in_specs=[
            pl.BlockSpec(memory_space=pltpu.VMEM),
            pl.BlockSpec(memory_space=pltpu.VMEM),
        ],
        out_specs=pl.BlockSpec(memory_space=pltpu.VMEM),
        scratch_shapes=[
            pltpu.VMEM((N_DEV, S, k), jnp.bfloat16),
            pltpu.SemaphoreType.DMA((N_DEV, H)),
            pltpu.SemaphoreType.DMA((N_DEV, H)),
            pltpu.SemaphoreType.DMA((N_DEV, H)),
            pltpu.SemaphoreType.DMA((N_DEV, H)),
        ],
        compiler_params=pltpu.CompilerParams(collective_id=0),
    )(tb, wb)
